# Initial kernel scaffold; baseline (speedup 1.0000x reference)
import jax
import jax.numpy as jnp
from jax.experimental import pallas as pl


def kernel(X, edge_index, edge_weight, W0, W1):
    raise NotImplementedError("write your pallas kernel here")


# speedup vs baseline: 2.5099x; 2.5099x over previous
import functools

import jax
import jax.numpy as jnp
from jax import lax
from jax.experimental import pallas as pl
from jax.experimental.pallas import tpu as pltpu
from jax.experimental.pallas import tpu_sc as plsc

NC = 2
NS = 16
BATCH = 128


def _make_spmm(n, d, e_pad):
  ept = e_pad // NS
  nch = ept // BATCH
  cr = 80
  ncopy = n // cr
  maxi = (ncopy + NS - 1) // NS

  mesh = plsc.VectorSubcoreMesh(core_axis_name="c", subcore_axis_name="s")

  @functools.partial(
      pl.kernel,
      out_type=jax.ShapeDtypeStruct((NC, n, d), jnp.float32),
      mesh=mesh,
      compiler_params=pltpu.CompilerParams(use_tc_tiling_on_sc=False),
      scratch_types=[
          pltpu.VMEM((BATCH,), jnp.int32),
          pltpu.VMEM((BATCH,), jnp.int32),
          pltpu.VMEM((BATCH,), jnp.float32),
          pltpu.VMEM((BATCH, d), jnp.float32),
          pltpu.VMEM_SHARED((n, d), jnp.float32),
          pltpu.SemaphoreType.DMA,
      ],
  )
  def spmm(h_hbm, col_hbm, row_hbm, w_hbm, out_hbm,
           cidx, ridx, wv, rows, acc, sem):
    c = lax.axis_index("c")
    s = lax.axis_index("s")

    zvec = jnp.zeros((16,), jnp.float32)

    def zfill(e, carry):
      for f in range(d // 16):
        rows[e, pl.ds(f * 16, 16)] = zvec
      return carry

    lax.fori_loop(0, BATCH, zfill, 0)
    for i in range(maxi):
      k = s + NS * i

      @pl.when(k < ncopy)
      def _():
        off = pl.multiple_of(k * cr, 8)
        pltpu.sync_copy(rows.at[pl.ds(0, cr)], acc.at[pl.ds(off, cr)])

    plsc.subcore_barrier()

    def chunk(j, carry):
      base = pl.multiple_of(s * ept + j * BATCH, BATCH)
      pltpu.sync_copy(col_hbm.at[pl.ds(base, BATCH)], cidx)
      pltpu.sync_copy(row_hbm.at[pl.ds(base, BATCH)], ridx)
      pltpu.sync_copy(w_hbm.at[pl.ds(base, BATCH)], wv)
      pltpu.async_copy(h_hbm.at[c].at[cidx], rows, sem).wait()

      def scale(g, carry2):
        wg = wv[pl.ds(g * 16, 16)]
        for j in range(16):
          w = wg[j]
          e = g * 16 + j
          for f in range(d // 16):
            sl = pl.ds(f * 16, 16)
            rows[e, sl] = rows[e, sl] * w
        return carry2

      lax.fori_loop(0, BATCH // 16, scale, 0)
      pltpu.sync_copy(rows, acc.at[ridx], add=True)
      return carry

    lax.fori_loop(0, nch, chunk, 0)
    plsc.subcore_barrier()

    for i in range(maxi):
      k = s + NS * i

      @pl.when(k < ncopy)
      def _():
        off = pl.multiple_of(k * cr, 8)
        pltpu.sync_copy(acc.at[pl.ds(off, cr)], out_hbm.at[c].at[pl.ds(off, cr)])

  return spmm


def _mm1(x, w0h):
  n, cdim = x.shape
  d = w0h.shape[1]
  bn = 1000

  def body(x_ref, w_ref, o_ref):
    o_ref[0] = lax.dot_general(
        x_ref[...], w_ref[0], (((1,), (1,)), ((), ())),
        preferred_element_type=jnp.float32)

  return pl.pallas_call(
      body,
      grid=(2, n // bn),
      in_specs=[
          pl.BlockSpec((bn, cdim), lambda c, i: (i, 0)),
          pl.BlockSpec((1, d, cdim), lambda c, i: (c, 0, 0)),
      ],
      out_specs=pl.BlockSpec((1, bn, d), lambda c, i: (c, i, 0)),
      out_shape=jax.ShapeDtypeStruct((2, n, d), jnp.float32),
  )(x, w0h)


def _mm2(s1, w1h):
  _, n, hhalf = s1.shape
  fh = w1h.shape[1]
  bn = 1000

  def body(s_ref, w_ref, o_ref):
    a = jnp.maximum(s_ref[0], 0.0)
    b = jnp.maximum(s_ref[1], 0.0)
    w = w_ref[0]
    o_ref[0] = (
        lax.dot_general(a, w[:, :hhalf], (((1,), (1,)), ((), ())),
                        preferred_element_type=jnp.float32)
        + lax.dot_general(b, w[:, hhalf:], (((1,), (1,)), ((), ())),
                          preferred_element_type=jnp.float32))

  return pl.pallas_call(
      body,
      grid=(2, n // bn),
      in_specs=[
          pl.BlockSpec((2, bn, hhalf), lambda c, i: (0, i, 0)),
          pl.BlockSpec((1, fh, 2 * hhalf), lambda c, i: (c, 0, 0)),
      ],
      out_specs=pl.BlockSpec((1, bn, fh), lambda c, i: (c, i, 0)),
      out_shape=jax.ShapeDtypeStruct((2, n, fh), jnp.float32),
  )(s1, w1h)


def _softmax(s2):
  _, n, fh = s2.shape
  bn = 1000

  def body(s_ref, o_ref):
    z = jnp.concatenate([s_ref[0], s_ref[1]], axis=-1)
    m = jnp.max(z, axis=-1, keepdims=True)
    ez = jnp.exp(z - m)
    o_ref[...] = ez / jnp.sum(ez, axis=-1, keepdims=True)

  return pl.pallas_call(
      body,
      grid=(n // bn,),
      in_specs=[pl.BlockSpec((2, bn, fh), lambda i: (0, i, 0))],
      out_specs=pl.BlockSpec((bn, 2 * fh), lambda i: (i, 0)),
      out_shape=jax.ShapeDtypeStruct((n, 2 * fh), jnp.float32),
  )(s2)


def kernel(X, edge_index, edge_weight, W0, W1):
  n, cdim = X.shape
  hdim = W0.shape[0]
  fdim = W1.shape[0]
  e = edge_weight.shape[0]

  gran = NS * BATCH
  e_pad = ((e + gran - 1) // gran) * gran
  pad = e_pad - e
  row = jnp.pad(edge_index[0], (0, pad))
  col = jnp.pad(edge_index[1], (0, pad))
  w = jnp.pad(edge_weight, (0, pad))

  w0h = W0.reshape(NC, hdim // NC, cdim)
  h1 = _mm1(X, w0h)
  s1 = _make_spmm(n, hdim // NC, e_pad)(h1, col, row, w)
  w1h = W1.reshape(NC, fdim // NC, hdim)
  z = _mm2(s1, w1h)
  s2 = _make_spmm(n, fdim // NC, e_pad)(z, col, row, w)
  return _softmax(s2)


# speedup vs baseline: 6.5052x; 2.5918x over previous
import functools

import jax
import jax.numpy as jnp
from jax import lax
from jax.experimental import pallas as pl
from jax.experimental.pallas import tpu as pltpu
from jax.experimental.pallas import tpu_sc as plsc

NC = 2
NS = 16
BATCH = 128


def _make_spmm(n, d, e_pad):
  ept = e_pad // NS
  nch = ept // BATCH
  cr = 80
  ncopy = n // cr
  maxi = (ncopy + NS - 1) // NS

  mesh = plsc.VectorSubcoreMesh(core_axis_name="c", subcore_axis_name="s")

  @functools.partial(
      pl.kernel,
      out_type=jax.ShapeDtypeStruct((NC, n, d), jnp.float32),
      mesh=mesh,
      compiler_params=pltpu.CompilerParams(use_tc_tiling_on_sc=False),
      scratch_types=[
          pltpu.VMEM((nch, BATCH), jnp.int32),
          pltpu.VMEM((nch, BATCH), jnp.int32),
          pltpu.VMEM((nch, BATCH), jnp.float32),
          pltpu.VMEM((BATCH, d), jnp.float32),
          pltpu.VMEM((BATCH, d), jnp.float32),
          pltpu.VMEM_SHARED((n, d), jnp.float32),
          pltpu.SemaphoreType.DMA,
          pltpu.SemaphoreType.DMA,
      ],
  )
  def spmm(h_hbm, col_hbm, row_hbm, w_hbm, out_hbm,
           col_v, row_v, w_v, rows0, rows1, acc, sem0, sem1):
    c = lax.axis_index("c")
    s = lax.axis_index("s")

    pltpu.sync_copy(col_hbm.at[s], col_v)
    pltpu.sync_copy(row_hbm.at[s], row_v)
    pltpu.sync_copy(w_hbm.at[s], w_v)

    zvec = jnp.zeros((16,), jnp.float32)

    def zfill(e, carry):
      for f in range(d // 16):
        rows0[e, pl.ds(f * 16, 16)] = zvec
      return carry

    lax.fori_loop(0, BATCH, zfill, 0)
    for i in range(maxi):
      k = s + NS * i

      @pl.when(k < ncopy)
      def _():
        off = pl.multiple_of(k * cr, 8)
        pltpu.sync_copy(rows0.at[pl.ds(0, cr)], acc.at[pl.ds(off, cr)])

    plsc.subcore_barrier()

    hsrc = h_hbm.at[c]

    def gather(j, buf, sm):
      pltpu.async_copy(hsrc.at[col_v.at[j]], buf, sm)

    def gwait(j, buf, sm):
      pltpu.make_async_copy(hsrc.at[col_v.at[j]], buf, sm).wait()

    def scale(buf, j):
      for g in range(BATCH // 16):
        wg = w_v[j, pl.ds(g * 16, 16)]
        for t in range(16):
          wt = wg[t]
          e = g * 16 + t
          for f in range(d // 16):
            sl = pl.ds(f * 16, 16)
            buf[e, sl] = buf[e, sl] * wt

    gather(0, rows0, sem0)

    def body(jj, carry):
      j0 = 2 * jj
      gather(j0 + 1, rows1, sem1)
      gwait(j0, rows0, sem0)
      scale(rows0, j0)
      pltpu.sync_copy(rows0, acc.at[row_v.at[j0]], add=True)

      @pl.when(j0 + 2 < nch)
      def _():
        gather(j0 + 2, rows0, sem0)

      gwait(j0 + 1, rows1, sem1)
      scale(rows1, j0 + 1)
      pltpu.sync_copy(rows1, acc.at[row_v.at[j0 + 1]], add=True)
      return carry

    lax.fori_loop(0, nch // 2, body, 0)
    plsc.subcore_barrier()

    for i in range(maxi):
      k = s + NS * i

      @pl.when(k < ncopy)
      def _():
        off = pl.multiple_of(k * cr, 8)
        pltpu.sync_copy(acc.at[pl.ds(off, cr)], out_hbm.at[c].at[pl.ds(off, cr)])

  return spmm


def _mm1(x, w0h):
  n, cdim = x.shape
  d = w0h.shape[1]
  bn = 1000

  def body(x_ref, w_ref, o_ref):
    o_ref[0] = lax.dot_general(
        x_ref[...], w_ref[0], (((1,), (1,)), ((), ())),
        preferred_element_type=jnp.float32)

  return pl.pallas_call(
      body,
      grid=(2, n // bn),
      in_specs=[
          pl.BlockSpec((bn, cdim), lambda c, i: (i, 0)),
          pl.BlockSpec((1, d, cdim), lambda c, i: (c, 0, 0)),
      ],
      out_specs=pl.BlockSpec((1, bn, d), lambda c, i: (c, i, 0)),
      out_shape=jax.ShapeDtypeStruct((2, n, d), jnp.float32),
  )(x, w0h)


def _mm2(s1, w1h):
  _, n, hhalf = s1.shape
  fh = w1h.shape[1]
  bn = 1000

  def body(s_ref, w_ref, o_ref):
    a = jnp.maximum(s_ref[0], 0.0)
    b = jnp.maximum(s_ref[1], 0.0)
    w = w_ref[0]
    o_ref[0] = (
        lax.dot_general(a, w[:, :hhalf], (((1,), (1,)), ((), ())),
                        preferred_element_type=jnp.float32)
        + lax.dot_general(b, w[:, hhalf:], (((1,), (1,)), ((), ())),
                          preferred_element_type=jnp.float32))

  return pl.pallas_call(
      body,
      grid=(2, n // bn),
      in_specs=[
          pl.BlockSpec((2, bn, hhalf), lambda c, i: (0, i, 0)),
          pl.BlockSpec((1, fh, 2 * hhalf), lambda c, i: (c, 0, 0)),
      ],
      out_specs=pl.BlockSpec((1, bn, fh), lambda c, i: (c, i, 0)),
      out_shape=jax.ShapeDtypeStruct((2, n, fh), jnp.float32),
  )(s1, w1h)


def _softmax(s2):
  _, n, fh = s2.shape
  bn = 1000

  def body(s_ref, o_ref):
    z = jnp.concatenate([s_ref[0], s_ref[1]], axis=-1)
    m = jnp.max(z, axis=-1, keepdims=True)
    ez = jnp.exp(z - m)
    o_ref[...] = ez / jnp.sum(ez, axis=-1, keepdims=True)

  return pl.pallas_call(
      body,
      grid=(n // bn,),
      in_specs=[pl.BlockSpec((2, bn, fh), lambda i: (0, i, 0))],
      out_specs=pl.BlockSpec((bn, 2 * fh), lambda i: (i, 0)),
      out_shape=jax.ShapeDtypeStruct((n, 2 * fh), jnp.float32),
  )(s2)


def kernel(X, edge_index, edge_weight, W0, W1):
  n, cdim = X.shape
  hdim = W0.shape[0]
  fdim = W1.shape[0]
  e = edge_weight.shape[0]

  gran = 2 * NS * BATCH
  e_pad = ((e + gran - 1) // gran) * gran
  pad = e_pad - e
  nch = e_pad // (NS * BATCH)
  row = jnp.pad(edge_index[0], (0, pad)).reshape(NS, nch, BATCH)
  col = jnp.pad(edge_index[1], (0, pad)).reshape(NS, nch, BATCH)
  w = jnp.pad(edge_weight, (0, pad)).reshape(NS, nch, BATCH)

  w0h = W0.reshape(NC, hdim // NC, cdim)
  h1 = _mm1(X, w0h)
  s1 = _make_spmm(n, hdim // NC, e_pad)(h1, col, row, w)
  w1h = W1.reshape(NC, fdim // NC, hdim)
  z = _mm2(s1, w1h)
  s2 = _make_spmm(n, fdim // NC, e_pad)(z, col, row, w)
  return _softmax(s2)
